# baseline (speedup 1.0000x reference)
import jax
import jax.numpy as jnp
from jax.experimental import pallas as pl
from jax.experimental.pallas import tpu as pltpu

_B = 16384
_C = 1000
_W = 2048
_HW = _W // 2
_NBLK = _B // _W
_NR = _C // 8
_K_RANK = int(_B * 0.3) + 1


def _sublane_reduce(v, op):
    for sh in (4, 2, 1):
        v = op(v, pltpu.roll(v, sh, 0))
    return v


def _ce_half(x_ref, t_ref, lo):
    t = t_ref[0, 0:1, lo:lo + _HW]
    iota8 = jax.lax.broadcasted_iota(jnp.int32, (8, _HW), 0)
    tm = jnp.broadcast_to(t, (8, _HW)) - iota8

    m = x_ref[0:8, lo:lo + _HW]
    for r in range(1, _NR):
        m = jnp.maximum(m, x_ref[8 * r:8 * r + 8, lo:lo + _HW])
    m = _sublane_reduce(m, jnp.maximum)

    s = jnp.zeros((8, _HW), jnp.float32)
    g = jnp.zeros((8, _HW), jnp.float32)
    for r in range(_NR):
        xr = x_ref[8 * r:8 * r + 8, lo:lo + _HW]
        s = s + jnp.exp(xr - m)
        g = g + jnp.where(tm == 8 * r, xr, 0.0)
    s = _sublane_reduce(s, jnp.add)
    g = _sublane_reduce(g, jnp.add)

    return (m + jnp.log(s) - g)[0:1, :]


def _ce_select_kernel(x_ref, t_ref, o_ref, ce_ref):
    i = pl.program_id(0)
    for h in range(2):
        ce = _ce_half(x_ref, t_ref, h * _HW)
        ce_ref[pl.ds(i, 1), h * _HW:(h + 1) * _HW] = ce

    @pl.when(i == _NBLK - 1)
    def _():
        ce_all = ce_ref[...]
        raw = jax.lax.bitcast_convert_type(ce_all, jnp.int32)
        keys = raw ^ ((raw >> 31) & jnp.int32(0x7FFFFFFF))
        nonneg = jnp.sum((keys >= 0).astype(jnp.int32))
        base0 = jnp.where(nonneg >= _K_RANK, jnp.int32(0),
                          jnp.int32(-2147483648))

        def body(b, base):
            cand = base + (jnp.int32(1) << (30 - b))
            cnt = jnp.sum((keys >= cand).astype(jnp.int32))
            return jnp.where(cnt >= _K_RANK, cand, base)

        lam = jax.lax.fori_loop(0, 31, body, base0)
        kept = jnp.where(keys >= lam, ce_all, 0.0)
        o_ref[0, 0] = jnp.sum(kept) / _B


def kernel(output, target):
    xt = output.T
    t3 = target.astype(jnp.int32).reshape(_NBLK, 1, _W)
    out = pl.pallas_call(
        _ce_select_kernel,
        grid=(_NBLK,),
        in_specs=[
            pl.BlockSpec((_C, _W), lambda i: (0, i)),
            pl.BlockSpec((1, 1, _W), lambda i: (i, 0, 0)),
        ],
        out_specs=pl.BlockSpec(memory_space=pltpu.SMEM),
        out_shape=jax.ShapeDtypeStruct((1, 1), jnp.float32),
        scratch_shapes=[pltpu.VMEM((_NBLK, _W), jnp.float32)],
    )(xt, t3)
    return out[0, 0]


# speedup vs baseline: 1.0328x; 1.0328x over previous
import jax
import jax.numpy as jnp
from jax.experimental import pallas as pl
from jax.experimental.pallas import tpu as pltpu

_B = 16384
_C = 1000
_W = 2048
_HW = _W // 2
_NBLK = _B // _W
_NR = _C // 8
_K_RANK = int(_B * 0.3) + 1


def _sublane_reduce(v, op):
    for sh in (4, 2, 1):
        v = op(v, pltpu.roll(v, sh, 0))
    return v


def _ce_half(x_ref, t_ref, lo):
    t = t_ref[0, 0:1, lo:lo + _HW]
    iota8 = jax.lax.broadcasted_iota(jnp.int32, (8, _HW), 0)
    tq = jnp.broadcast_to(t >> 3, (8, _HW))
    ts = jnp.broadcast_to(t & 7, (8, _HW))

    m = x_ref[0:8, lo:lo + _HW]
    for r in range(1, _NR):
        m = jnp.maximum(m, x_ref[8 * r:8 * r + 8, lo:lo + _HW])
    m = _sublane_reduce(m, jnp.maximum)

    s = jnp.zeros((8, _HW), jnp.float32)
    z = jnp.zeros((8, _HW), jnp.float32)
    for r in range(_NR):
        xr = x_ref[8 * r:8 * r + 8, lo:lo + _HW]
        s = s + jnp.exp(xr - m)
        z = jnp.where(tq == r, xr, z)
    s = _sublane_reduce(s, jnp.add)
    g = _sublane_reduce(jnp.where(iota8 == ts, z, 0.0), jnp.add)

    return (m + jnp.log(s) - g)[0:1, :]


def _ce_select_kernel(x_ref, t_ref, o_ref, ce_ref):
    i = pl.program_id(0)
    for h in range(2):
        ce = _ce_half(x_ref, t_ref, h * _HW)
        ce_ref[pl.ds(i, 1), h * _HW:(h + 1) * _HW] = ce

    @pl.when(i == _NBLK - 1)
    def _():
        ce_all = ce_ref[...]
        raw = jax.lax.bitcast_convert_type(ce_all, jnp.int32)
        keys = raw ^ ((raw >> 31) & jnp.int32(0x7FFFFFFF))
        nonneg = jnp.sum((keys >= 0).astype(jnp.int32))
        base0 = jnp.where(nonneg >= _K_RANK, jnp.int32(0),
                          jnp.int32(-2147483648))

        def body(b, base):
            cand = base + (jnp.int32(1) << (30 - b))
            cnt = jnp.sum((keys >= cand).astype(jnp.int32))
            return jnp.where(cnt >= _K_RANK, cand, base)

        lam = jax.lax.fori_loop(0, 31, body, base0)
        kept = jnp.where(keys >= lam, ce_all, 0.0)
        o_ref[0, 0] = jnp.sum(kept) / _B


def kernel(output, target):
    xt = output.T
    t3 = target.astype(jnp.int32).reshape(_NBLK, 1, _W)
    out = pl.pallas_call(
        _ce_select_kernel,
        grid=(_NBLK,),
        in_specs=[
            pl.BlockSpec((_C, _W), lambda i: (0, i)),
            pl.BlockSpec((1, 1, _W), lambda i: (i, 0, 0)),
        ],
        out_specs=pl.BlockSpec(memory_space=pltpu.SMEM),
        out_shape=jax.ShapeDtypeStruct((1, 1), jnp.float32),
        scratch_shapes=[pltpu.VMEM((_NBLK, _W), jnp.float32)],
    )(xt, t3)
    return out[0, 0]


# speedup vs baseline: 1.0349x; 1.0020x over previous
import jax
import jax.numpy as jnp
from jax.experimental import pallas as pl
from jax.experimental.pallas import tpu as pltpu

_B = 16384
_C = 1000
_W = 2048
_HW = _W // 2
_NBLK = _B // _W
_NR = _C // 8
_K_RANK = int(_B * 0.3) + 1


def _sublane_reduce(v, op):
    for sh in (4, 2, 1):
        v = op(v, pltpu.roll(v, sh, 0))
    return v


def _ce_half(x_ref, t_ref, lo):
    t = t_ref[0, 0:1, lo:lo + _HW]
    iota8 = jax.lax.broadcasted_iota(jnp.int32, (8, _HW), 0)
    tq = jnp.broadcast_to(t >> 3, (8, _HW))
    ts = jnp.broadcast_to(t & 7, (8, _HW))

    m = x_ref[0:8, lo:lo + _HW]
    for r in range(1, _NR):
        m = jnp.maximum(m, x_ref[8 * r:8 * r + 8, lo:lo + _HW])
    m = _sublane_reduce(m, jnp.maximum)

    s0 = jnp.zeros((8, _HW), jnp.float32)
    s1 = jnp.zeros((8, _HW), jnp.float32)
    z0 = jnp.zeros((8, _HW), jnp.float32)
    z1 = jnp.zeros((8, _HW), jnp.float32)
    for r in range(0, _NR - 1, 2):
        xr = x_ref[8 * r:8 * r + 8, lo:lo + _HW]
        s0 = s0 + jnp.exp(xr - m)
        z0 = jnp.where(tq == r, xr, z0)
        xr2 = x_ref[8 * r + 8:8 * r + 16, lo:lo + _HW]
        s1 = s1 + jnp.exp(xr2 - m)
        z1 = jnp.where(tq == r + 1, xr2, z1)
    if _NR % 2:
        xr = x_ref[8 * (_NR - 1):8 * _NR, lo:lo + _HW]
        s0 = s0 + jnp.exp(xr - m)
        z0 = jnp.where(tq == _NR - 1, xr, z0)
    s = _sublane_reduce(s0 + s1, jnp.add)
    z = z0 + z1
    g = _sublane_reduce(jnp.where(iota8 == ts, z, 0.0), jnp.add)

    return (m + jnp.log(s) - g)[0:1, :]


def _ce_select_kernel(x_ref, t_ref, o_ref, ce_ref):
    i = pl.program_id(0)
    for h in range(2):
        ce = _ce_half(x_ref, t_ref, h * _HW)
        ce_ref[pl.ds(i, 1), h * _HW:(h + 1) * _HW] = ce

    @pl.when(i == _NBLK - 1)
    def _():
        ce_all = ce_ref[...]
        raw = jax.lax.bitcast_convert_type(ce_all, jnp.int32)
        keys = raw ^ ((raw >> 31) & jnp.int32(0x7FFFFFFF))
        nonneg = jnp.sum((keys >= 0).astype(jnp.int32))
        base0 = jnp.where(nonneg >= _K_RANK, jnp.int32(0),
                          jnp.int32(-2147483648))

        def body(b, base):
            cand = base + (jnp.int32(1) << (30 - b))
            cnt = jnp.sum((keys >= cand).astype(jnp.int32))
            return jnp.where(cnt >= _K_RANK, cand, base)

        lam = jax.lax.fori_loop(0, 31, body, base0)
        kept = jnp.where(keys >= lam, ce_all, 0.0)
        o_ref[0, 0] = jnp.sum(kept) / _B


def kernel(output, target):
    xt = output.T
    t3 = target.astype(jnp.int32).reshape(_NBLK, 1, _W)
    out = pl.pallas_call(
        _ce_select_kernel,
        grid=(_NBLK,),
        in_specs=[
            pl.BlockSpec((_C, _W), lambda i: (0, i)),
            pl.BlockSpec((1, 1, _W), lambda i: (i, 0, 0)),
        ],
        out_specs=pl.BlockSpec(memory_space=pltpu.SMEM),
        out_shape=jax.ShapeDtypeStruct((1, 1), jnp.float32),
        scratch_shapes=[pltpu.VMEM((_NBLK, _W), jnp.float32)],
    )(xt, t3)
    return out[0, 0]
